# baseline (speedup 1.0000x reference)
import functools

import jax
import jax.numpy as jnp
from jax import lax
from jax.experimental import pallas as pl
from jax.experimental.pallas import tpu as pltpu
from jax.experimental.pallas import tpu_sc as plsc

E = 8
K = 2
D = 768
H = 2048
T = 2048
B = 128
NT = T * K // B + E
NSLOT = NT * B

_SC_NC = 2
_SC_NS = 16
_NW = _SC_NC * _SC_NS


def _book_body(idx_ref, p0_ref, p1_ref, meta_ref):
    e0 = idx_ref[:, 0:1]
    e1 = idx_ref[:, 1:2]
    lane8 = lax.broadcasted_iota(jnp.int32, (1, E), 1)
    oh0 = (e0 == lane8).astype(jnp.int32)
    oh1 = (e1 == lane8).astype(jnp.int32)
    c = jnp.concatenate([oh0, oh1], axis=1)
    s = 1
    while s < T:
        c = c + jnp.concatenate(
            [jnp.zeros((s, 2 * E), jnp.int32), c[:-s, :]], axis=0)
        s *= 2
    rank0 = jnp.sum(c[:, :E] * oh0, axis=1, keepdims=True) - 1
    rank1 = jnp.sum(c[:, E:] * oh1, axis=1, keepdims=True) - 1
    cnt = c[T - 1:T, :]
    cnt0 = cnt[:, :E]
    counts = cnt[:, :E] + cnt[:, E:]
    tiles_e = (counts + B - 1) // B
    tri = (lax.broadcasted_iota(jnp.int32, (E, E), 0)
           <= lax.broadcasted_iota(jnp.int32, (E, E), 1)).astype(jnp.float32)
    ts_incl = lax.dot_general(
        tiles_e.astype(jnp.float32), tri,
        dimension_numbers=(((1,), (0,)), ((), ()))).astype(jnp.int32)
    seg_start = (ts_incl - tiles_e) * B
    sb0 = jnp.sum(oh0 * seg_start, axis=1, keepdims=True)
    sb1 = jnp.sum(oh1 * seg_start, axis=1, keepdims=True)
    c0e1 = jnp.sum(oh1 * cnt0, axis=1, keepdims=True)
    p0_ref[...] = sb0 + rank0
    p1_ref[...] = sb1 + c0e1 + rank1
    eye = (lax.broadcasted_iota(jnp.int32, (E, E), 0)
           == lax.broadcasted_iota(jnp.int32, (E, E), 1)).astype(jnp.float32)
    ts_col = lax.dot_general(
        eye, ts_incl.astype(jnp.float32),
        dimension_numbers=(((1,), (1,)), ((), ()))).astype(jnp.int32)
    tlane = lax.broadcasted_iota(jnp.int32, (1, 128), 1)
    te_row = jnp.sum((tlane >= ts_col).astype(jnp.int32), axis=0,
                     keepdims=True)
    te_row = jnp.minimum(te_row, E - 1)
    total = ts_col[E - 1:E, 0:1]
    used_row = (tlane < total).astype(jnp.int32)
    meta_ref[...] = jnp.concatenate(
        [te_row, used_row, jnp.zeros((6, 128), jnp.int32)], axis=0)


def _bookkeeping(idx):
    return pl.pallas_call(
        _book_body,
        out_shape=(
            jax.ShapeDtypeStruct((T, 1), jnp.int32),
            jax.ShapeDtypeStruct((T, 1), jnp.int32),
            jax.ShapeDtypeStruct((8, 128), jnp.int32),
        ),
    )(idx)


def _sc_dispatch(xf, p0, p1, w0, w1):
    per_w = T // _NW
    mesh = plsc.VectorSubcoreMesh(core_axis_name="c", subcore_axis_name="s")

    @functools.partial(
        pl.kernel, mesh=mesh,
        out_type=(
            jax.ShapeDtypeStruct((NSLOT, D), jnp.float32),
            jax.ShapeDtypeStruct((NSLOT, 128), jnp.float32),
        ),
        scratch_types=[
            pltpu.VMEM((per_w, D), jnp.float32),
            pltpu.VMEM((per_w,), jnp.int32),
            pltpu.VMEM((per_w,), jnp.int32),
            pltpu.VMEM((per_w, 128), jnp.float32),
            pltpu.VMEM((per_w, 128), jnp.float32),
            pltpu.SemaphoreType.DMA,
            pltpu.SemaphoreType.DMA,
        ],
    )
    def k(x_hbm, p0_hbm, p1_hbm, w0_hbm, w1_hbm, xs_hbm, sw_hbm,
          xr, i0, i1, wb0, wb1, sem_x, sem_w):
        wid = lax.axis_index("s") * _SC_NC + lax.axis_index("c")
        base = wid * per_w
        pltpu.sync_copy(p0_hbm.at[pl.ds(base, per_w)], i0)
        pltpu.sync_copy(p1_hbm.at[pl.ds(base, per_w)], i1)
        pltpu.sync_copy(w0_hbm.at[pl.ds(base, per_w)], wb0)
        pltpu.sync_copy(w1_hbm.at[pl.ds(base, per_w)], wb1)
        pltpu.sync_copy(x_hbm.at[pl.ds(base, per_w)], xr)
        h0 = pltpu.async_copy(xr, xs_hbm.at[i0], sem_x)
        h1 = pltpu.async_copy(xr, xs_hbm.at[i1], sem_x)
        h2 = pltpu.async_copy(wb0, sw_hbm.at[i0], sem_w)
        h3 = pltpu.async_copy(wb1, sw_hbm.at[i1], sem_w)
        h0.wait()
        h1.wait()
        h2.wait()
        h3.wait()

    return k(xf, p0, p1, w0, w1)


def _sc_combine(ys, p0, p1):
    per_w = T // _NW
    ch = 32
    nch = per_w // ch
    mesh = plsc.VectorSubcoreMesh(core_axis_name="c", subcore_axis_name="s")

    @functools.partial(
        pl.kernel, mesh=mesh,
        out_type=jax.ShapeDtypeStruct((T, D), jnp.float32),
        scratch_types=[
            pltpu.VMEM((ch,), jnp.int32),
            pltpu.VMEM((ch,), jnp.int32),
            pltpu.VMEM((ch, D), jnp.float32),
            pltpu.VMEM((ch, D), jnp.float32),
            pltpu.SemaphoreType.DMA,
        ],
    )
    def k(ys_hbm, p0_hbm, p1_hbm, out_hbm, i0_v, i1_v, b0, b1, sem):
        wid = lax.axis_index("s") * _SC_NC + lax.axis_index("c")
        base = wid * per_w
        for c in range(nch):
            off = base + c * ch
            pltpu.sync_copy(p0_hbm.at[pl.ds(off, ch)], i0_v)
            pltpu.sync_copy(p1_hbm.at[pl.ds(off, ch)], i1_v)
            pltpu.async_copy(ys_hbm.at[i0_v], b0, sem).wait()
            pltpu.async_copy(ys_hbm.at[i1_v], b1, sem).wait()

            def row_add(r, _):
                for j in range(D // 16):
                    sl = pl.ds(j * 16, 16)
                    b0[r, sl] = b0[r, sl] + b1[r, sl]
                return _

            lax.fori_loop(0, ch, row_add, 0)
            pltpu.sync_copy(b0, out_hbm.at[pl.ds(off, ch)])

    return k(ys, p0, p1)


def _ffn_body(meta_ref, xs_ref, w1_ref, w3_ref, w2_ref, sw_ref, out_ref):
    t = pl.program_id(0)

    @pl.when(meta_ref[1, t] > 0)
    def _():
        x = xs_ref[...].astype(jnp.bfloat16)
        w1 = w1_ref[0].astype(jnp.bfloat16)
        w3 = w3_ref[0].astype(jnp.bfloat16)
        h1 = jnp.dot(x, w1, preferred_element_type=jnp.float32)
        h3 = jnp.dot(x, w3, preferred_element_type=jnp.float32)
        hid = (h1 * jax.nn.sigmoid(h1)) * h3
        y = jnp.dot(hid.astype(jnp.bfloat16), w2_ref[0].astype(jnp.bfloat16),
                    preferred_element_type=jnp.float32)
        out_ref[...] = y * sw_ref[:, 0:1]


def _ffn(xs, w1, w3, w2, sw, meta):
    grid_spec = pltpu.PrefetchScalarGridSpec(
        num_scalar_prefetch=1,
        grid=(NT,),
        in_specs=[
            pl.BlockSpec((B, D), lambda t, meta: (t, 0)),
            pl.BlockSpec((1, D, H), lambda t, meta: (meta[0, t], 0, 0)),
            pl.BlockSpec((1, D, H), lambda t, meta: (meta[0, t], 0, 0)),
            pl.BlockSpec((1, H, D), lambda t, meta: (meta[0, t], 0, 0)),
            pl.BlockSpec((B, 128), lambda t, meta: (t, 0)),
        ],
        out_specs=pl.BlockSpec((B, D), lambda t, meta: (t, 0)),
    )
    return pl.pallas_call(
        _ffn_body,
        grid_spec=grid_spec,
        out_shape=jax.ShapeDtypeStruct((NSLOT, D), jnp.float32),
    )(meta, xs, w1, w3, w2, sw)


def kernel(x, Wg, W1, W2, W3):
    orig_shape = x.shape
    xf = x.reshape(-1, x.shape[-1])

    scores = xf @ Wg
    vals, idx = lax.top_k(scores, K)
    w = jax.nn.softmax(vals, axis=-1)

    p0, p1, meta = _bookkeeping(idx.astype(jnp.int32))
    p0 = p0.reshape(T)
    p1 = p1.reshape(T)

    w128 = jnp.broadcast_to(w[:, :, None], (T, K, 128))
    xs, sw = _sc_dispatch(xf, p0, p1, w128[:, 0], w128[:, 1])
    ys = _ffn(xs, W1, W3, W2, sw, meta)
    y = _sc_combine(ys, p0, p1)
    return y.reshape(orig_shape)


# speedup vs baseline: 1.0570x; 1.0570x over previous
import functools

import jax
import jax.numpy as jnp
from jax import lax
from jax.experimental import pallas as pl
from jax.experimental.pallas import tpu as pltpu
from jax.experimental.pallas import tpu_sc as plsc

E = 8
K = 2
D = 768
H = 2048
T = 2048
B = 256
NT = T * K // B + E
NSLOT = NT * B

_SC_NC = 2
_SC_NS = 16
_NW = _SC_NC * _SC_NS


def _book_body(idx_ref, p0_ref, p1_ref, meta_ref):
    e0 = idx_ref[:, 0:1]
    e1 = idx_ref[:, 1:2]
    lane8 = lax.broadcasted_iota(jnp.int32, (1, E), 1)
    oh0 = (e0 == lane8).astype(jnp.int32)
    oh1 = (e1 == lane8).astype(jnp.int32)
    c = jnp.concatenate([oh0, oh1], axis=1)
    s = 1
    while s < T:
        c = c + jnp.concatenate(
            [jnp.zeros((s, 2 * E), jnp.int32), c[:-s, :]], axis=0)
        s *= 2
    rank0 = jnp.sum(c[:, :E] * oh0, axis=1, keepdims=True) - 1
    rank1 = jnp.sum(c[:, E:] * oh1, axis=1, keepdims=True) - 1
    cnt = c[T - 1:T, :]
    cnt0 = cnt[:, :E]
    counts = cnt[:, :E] + cnt[:, E:]
    tiles_e = (counts + B - 1) // B
    tri = (lax.broadcasted_iota(jnp.int32, (E, E), 0)
           <= lax.broadcasted_iota(jnp.int32, (E, E), 1)).astype(jnp.float32)
    ts_incl = lax.dot_general(
        tiles_e.astype(jnp.float32), tri,
        dimension_numbers=(((1,), (0,)), ((), ()))).astype(jnp.int32)
    seg_start = (ts_incl - tiles_e) * B
    sb0 = jnp.sum(oh0 * seg_start, axis=1, keepdims=True)
    sb1 = jnp.sum(oh1 * seg_start, axis=1, keepdims=True)
    c0e1 = jnp.sum(oh1 * cnt0, axis=1, keepdims=True)
    p0_ref[...] = sb0 + rank0
    p1_ref[...] = sb1 + c0e1 + rank1
    eye = (lax.broadcasted_iota(jnp.int32, (E, E), 0)
           == lax.broadcasted_iota(jnp.int32, (E, E), 1)).astype(jnp.float32)
    ts_col = lax.dot_general(
        eye, ts_incl.astype(jnp.float32),
        dimension_numbers=(((1,), (1,)), ((), ()))).astype(jnp.int32)
    tlane = lax.broadcasted_iota(jnp.int32, (1, 128), 1)
    te_row = jnp.sum((tlane >= ts_col).astype(jnp.int32), axis=0,
                     keepdims=True)
    te_row = jnp.minimum(te_row, E - 1)
    total = ts_col[E - 1:E, 0:1]
    used_row = (tlane < total).astype(jnp.int32)
    meta_ref[...] = jnp.concatenate(
        [te_row, used_row, jnp.zeros((6, 128), jnp.int32)], axis=0)


def _bookkeeping(idx):
    return pl.pallas_call(
        _book_body,
        out_shape=(
            jax.ShapeDtypeStruct((T, 1), jnp.int32),
            jax.ShapeDtypeStruct((T, 1), jnp.int32),
            jax.ShapeDtypeStruct((8, 128), jnp.int32),
        ),
    )(idx)


def _sc_dispatch(xf, p0, p1, w0, w1):
    per_w = T // _NW
    mesh = plsc.VectorSubcoreMesh(core_axis_name="c", subcore_axis_name="s")

    @functools.partial(
        pl.kernel, mesh=mesh,
        out_type=(
            jax.ShapeDtypeStruct((NSLOT, D), jnp.float32),
            jax.ShapeDtypeStruct((NSLOT, 128), jnp.float32),
        ),
        scratch_types=[
            pltpu.VMEM((per_w, D), jnp.float32),
            pltpu.VMEM((per_w,), jnp.int32),
            pltpu.VMEM((per_w,), jnp.int32),
            pltpu.VMEM((per_w, 128), jnp.float32),
            pltpu.VMEM((per_w, 128), jnp.float32),
            pltpu.SemaphoreType.DMA,
            pltpu.SemaphoreType.DMA,
        ],
    )
    def k(x_hbm, p0_hbm, p1_hbm, w0_hbm, w1_hbm, xs_hbm, sw_hbm,
          xr, i0, i1, wb0, wb1, sem_x, sem_w):
        wid = lax.axis_index("s") * _SC_NC + lax.axis_index("c")
        base = wid * per_w
        pltpu.sync_copy(p0_hbm.at[pl.ds(base, per_w)], i0)
        pltpu.sync_copy(p1_hbm.at[pl.ds(base, per_w)], i1)
        pltpu.sync_copy(w0_hbm.at[pl.ds(base, per_w)], wb0)
        pltpu.sync_copy(w1_hbm.at[pl.ds(base, per_w)], wb1)
        pltpu.sync_copy(x_hbm.at[pl.ds(base, per_w)], xr)
        h0 = pltpu.async_copy(xr, xs_hbm.at[i0], sem_x)
        h1 = pltpu.async_copy(xr, xs_hbm.at[i1], sem_x)
        h2 = pltpu.async_copy(wb0, sw_hbm.at[i0], sem_w)
        h3 = pltpu.async_copy(wb1, sw_hbm.at[i1], sem_w)
        h0.wait()
        h1.wait()
        h2.wait()
        h3.wait()

    return k(xf, p0, p1, w0, w1)


def _sc_combine(ys, p0, p1):
    per_w = T // _NW
    ch = 32
    nch = per_w // ch
    mesh = plsc.VectorSubcoreMesh(core_axis_name="c", subcore_axis_name="s")

    @functools.partial(
        pl.kernel, mesh=mesh,
        out_type=jax.ShapeDtypeStruct((T, D), jnp.float32),
        scratch_types=[
            pltpu.VMEM((ch,), jnp.int32),
            pltpu.VMEM((ch,), jnp.int32),
            pltpu.VMEM((ch, D), jnp.float32),
            pltpu.VMEM((ch, D), jnp.float32),
            pltpu.SemaphoreType.DMA,
        ],
    )
    def k(ys_hbm, p0_hbm, p1_hbm, out_hbm, i0_v, i1_v, b0, b1, sem):
        wid = lax.axis_index("s") * _SC_NC + lax.axis_index("c")
        base = wid * per_w
        for c in range(nch):
            off = base + c * ch
            pltpu.sync_copy(p0_hbm.at[pl.ds(off, ch)], i0_v)
            pltpu.sync_copy(p1_hbm.at[pl.ds(off, ch)], i1_v)
            pltpu.async_copy(ys_hbm.at[i0_v], b0, sem).wait()
            pltpu.async_copy(ys_hbm.at[i1_v], b1, sem).wait()

            def row_add(r, _):
                for j in range(D // 16):
                    sl = pl.ds(j * 16, 16)
                    b0[r, sl] = b0[r, sl] + b1[r, sl]
                return _

            lax.fori_loop(0, ch, row_add, 0)
            pltpu.sync_copy(b0, out_hbm.at[pl.ds(off, ch)])

    return k(ys, p0, p1)


def _ffn_body(meta_ref, xs_ref, w1_ref, w3_ref, w2_ref, sw_ref, out_ref):
    t = pl.program_id(0)

    @pl.when(meta_ref[1, t] > 0)
    def _():
        x = xs_ref[...].astype(jnp.bfloat16)
        w1 = w1_ref[0].astype(jnp.bfloat16)
        w3 = w3_ref[0].astype(jnp.bfloat16)
        h1 = jnp.dot(x, w1, preferred_element_type=jnp.float32)
        h3 = jnp.dot(x, w3, preferred_element_type=jnp.float32)
        hid = (h1 * jax.nn.sigmoid(h1)) * h3
        y = jnp.dot(hid.astype(jnp.bfloat16), w2_ref[0].astype(jnp.bfloat16),
                    preferred_element_type=jnp.float32)
        out_ref[...] = y * sw_ref[:, 0:1]


def _ffn(xs, w1, w3, w2, sw, meta):
    grid_spec = pltpu.PrefetchScalarGridSpec(
        num_scalar_prefetch=1,
        grid=(NT,),
        in_specs=[
            pl.BlockSpec((B, D), lambda t, meta: (t, 0)),
            pl.BlockSpec((1, D, H), lambda t, meta: (meta[0, t], 0, 0)),
            pl.BlockSpec((1, D, H), lambda t, meta: (meta[0, t], 0, 0)),
            pl.BlockSpec((1, H, D), lambda t, meta: (meta[0, t], 0, 0)),
            pl.BlockSpec((B, 128), lambda t, meta: (t, 0)),
        ],
        out_specs=pl.BlockSpec((B, D), lambda t, meta: (t, 0)),
    )
    return pl.pallas_call(
        _ffn_body,
        grid_spec=grid_spec,
        out_shape=jax.ShapeDtypeStruct((NSLOT, D), jnp.float32),
    )(meta, xs, w1, w3, w2, sw)


def kernel(x, Wg, W1, W2, W3):
    orig_shape = x.shape
    xf = x.reshape(-1, x.shape[-1])

    scores = xf @ Wg
    vals, idx = lax.top_k(scores, K)
    w = jax.nn.softmax(vals, axis=-1)

    p0, p1, meta = _bookkeeping(idx.astype(jnp.int32))
    p0 = p0.reshape(T)
    p1 = p1.reshape(T)

    w128 = jnp.broadcast_to(w[:, :, None], (T, K, 128))
    xs, sw = _sc_dispatch(xf, p0, p1, w128[:, 0], w128[:, 1])
    ys = _ffn(xs, W1, W3, W2, sw, meta)
    y = _sc_combine(ys, p0, p1)
    return y.reshape(orig_shape)
